# Initial kernel scaffold; baseline (speedup 1.0000x reference)
import functools

import jax
import jax.numpy as jnp
from jax import lax
from jax.experimental import pallas as pl
from jax.experimental.pallas import tpu as pltpu
from jax.experimental.pallas import tpu_sc as plsc

N = 10000
E = 320000
D_IN = 128
D = 32

NC = 2
NS = 16
NW = NC * NS
EPW = E // NW

CH = 128
NCH = (EPW + CH - 1) // CH
E_PAD = NW * NCH * CH
STRIPE = 626
N_PAD = NS * STRIPE

_MESH = plsc.VectorSubcoreMesh(core_axis_name="c", subcore_axis_name="s")


@functools.partial(
    pl.kernel,
    out_type=jax.ShapeDtypeStruct((NW, N), jnp.float32),
    mesh=_MESH,
    scratch_types=[
        pltpu.VMEM((EPW,), jnp.int32),
        pltpu.VMEM((N,), jnp.float32),
    ],
)
def _deg_kernel(dst_hbm, out_hbm, idx_v, acc_v):
    c = lax.axis_index("c")
    s = lax.axis_index("s")
    w = c * NS + s
    pltpu.sync_copy(dst_hbm.at[w], idx_v)
    zeros16 = jnp.zeros((16,), jnp.float32)
    ones16 = jnp.ones((16,), jnp.float32)

    def zero(i, _):
        acc_v[pl.ds(i * 16, 16)] = zeros16
        return 0

    lax.fori_loop(0, N // 16, zero, 0)

    def scat(i, _):
        idx = idx_v[pl.ds(i * 16, 16)]
        plsc.addupdate_scatter(acc_v, [idx], ones16)
        return 0

    lax.fori_loop(0, EPW // 16, scat, 0)
    pltpu.sync_copy(acc_v, out_hbm.at[w])


@functools.partial(
    pl.kernel,
    out_type=jax.ShapeDtypeStruct((NC, N_PAD, D), jnp.float32),
    mesh=_MESH,
    scratch_types=[
        pltpu.VMEM((CH,), jnp.int32),
        pltpu.VMEM((CH,), jnp.int32),
        pltpu.VMEM((CH, D), jnp.float32),
        pltpu.VMEM((STRIPE, D), jnp.float32),
        pltpu.VMEM_SHARED((N_PAD, D), jnp.float32),
        pltpu.SemaphoreType.DMA,
    ],
)
def _scatter_kernel(g_hbm, src_hbm, dst_hbm, out_hbm,
                    idx_s, idx_d, rows_v, buf_v, acc_sh, sem):
    c = lax.axis_index("c")
    s = lax.axis_index("s")
    w = c * NS + s
    zeros16 = jnp.zeros((16,), jnp.float32)

    def zero(j, _):
        buf_v[j, pl.ds(0, 16)] = zeros16
        buf_v[j, pl.ds(16, 16)] = zeros16
        return 0

    lax.fori_loop(0, STRIPE, zero, 0)
    pltpu.sync_copy(buf_v, acc_sh.at[pl.ds(s * STRIPE, STRIPE)])
    plsc.subcore_barrier()

    def edge_chunk(j, _):
        pltpu.sync_copy(src_hbm.at[w, j], idx_s)
        pltpu.sync_copy(dst_hbm.at[w, j], idx_d)
        pltpu.async_copy(g_hbm.at[idx_s], rows_v, sem).wait()
        pltpu.sync_copy(rows_v, acc_sh.at[idx_d], add=True)
        return 0

    lax.fori_loop(0, NCH, edge_chunk, 0)
    plsc.subcore_barrier()
    pltpu.sync_copy(acc_sh.at[pl.ds(s * STRIPE, STRIPE)], buf_v)
    pltpu.sync_copy(buf_v, out_hbm.at[c, pl.ds(s * STRIPE, STRIPE)])


BLK = 400


def _g_body(x_ref, w1_ref, degp_ref, g_ref):
    deg = jnp.sum(degp_ref[...], axis=0) + 1.0
    dis = lax.rsqrt(deg)
    h = jnp.dot(x_ref[...], w1_ref[...], preferred_element_type=jnp.float32)
    g_ref[...] = h * dis[:, None]


def _g_table(x, w1, deg_p):
    return pl.pallas_call(
        _g_body,
        grid=(N // BLK,),
        in_specs=[
            pl.BlockSpec((BLK, D_IN), lambda i: (i, 0)),
            pl.BlockSpec((D_IN, D), lambda i: (0, 0)),
            pl.BlockSpec((NW, BLK), lambda i: (0, i)),
        ],
        out_specs=pl.BlockSpec((BLK, D), lambda i: (i, 0)),
        out_shape=jax.ShapeDtypeStruct((N, D), jnp.float32),
    )(x, w1, deg_p)


def _out_body(s0_ref, s1_ref, g_ref, degp_ref, b1_ref, w2_ref, b2_ref, o_ref):
    deg = jnp.sum(degp_ref[...], axis=0) + 1.0
    dis = lax.rsqrt(deg)
    agg = (s0_ref[...] + s1_ref[...] + g_ref[...]) * dis[:, None]
    a = jnp.maximum(agg + b1_ref[...], 0.0)
    o = jnp.dot(a, w2_ref[...], preferred_element_type=jnp.float32)
    o_ref[...] = jnp.maximum(o + b2_ref[...], 0.0)


def _final(s0, s1, g, deg_p, b1, w2, b2):
    return pl.pallas_call(
        _out_body,
        grid=(N // BLK,),
        in_specs=[
            pl.BlockSpec((BLK, D), lambda i: (i, 0)),
            pl.BlockSpec((BLK, D), lambda i: (i, 0)),
            pl.BlockSpec((BLK, D), lambda i: (i, 0)),
            pl.BlockSpec((NW, BLK), lambda i: (0, i)),
            pl.BlockSpec((1, D), lambda i: (0, 0)),
            pl.BlockSpec((D, D), lambda i: (0, 0)),
            pl.BlockSpec((1, D), lambda i: (0, 0)),
        ],
        out_specs=pl.BlockSpec((BLK, D), lambda i: (i, 0)),
        out_shape=jax.ShapeDtypeStruct((N, D), jnp.float32),
    )(s0, s1, g, deg_p, b1, w2, b2)


def kernel(x, edge_index, W_gcn, b_gcn, W_dense, b_dense):
    src = edge_index[0].astype(jnp.int32)
    dst = edge_index[1].astype(jnp.int32)

    deg_p = _deg_kernel(dst.reshape(NW, EPW))

    g = _g_table(x, W_gcn, deg_p)

    pad = E_PAD - E
    src_p = jnp.concatenate([src, jnp.zeros((pad,), jnp.int32)])
    dst_p = jnp.concatenate([dst, jnp.full((pad,), N, jnp.int32)])
    s_all = _scatter_kernel(
        g, src_p.reshape(NW, NCH, CH), dst_p.reshape(NW, NCH, CH))

    s0 = s_all[0, :N, :]
    s1 = s_all[1, :N, :]
    return _final(s0, s1, g, deg_p,
                  b_gcn.reshape(1, D), W_dense, b_dense.reshape(1, D))


# speedup vs baseline: 22.3845x; 22.3845x over previous
import functools

import jax
import jax.numpy as jnp
from jax import lax
from jax.experimental import pallas as pl
from jax.experimental.pallas import tpu as pltpu
from jax.experimental.pallas import tpu_sc as plsc

N = 10000
E = 320000
D_IN = 128
D = 32

NC = 2
NS = 16
NW = NC * NS
EPW = E // NW

CH = 128
NCH = (EPW + CH - 1) // CH
E_PAD = NW * NCH * CH
STRIPE = 632
N_PAD = NS * STRIPE

_MESH = plsc.VectorSubcoreMesh(core_axis_name="c", subcore_axis_name="s")


DW = 16


@functools.partial(
    pl.kernel,
    out_type=jax.ShapeDtypeStruct((NC, N_PAD, DW), jnp.float32),
    mesh=_MESH,
    scratch_types=[
        pltpu.VMEM((CH,), jnp.int32),
        pltpu.VMEM((CH, DW), jnp.float32),
        pltpu.VMEM((STRIPE, DW), jnp.float32),
        pltpu.VMEM_SHARED((N_PAD, DW), jnp.float32),
    ],
    compiler_params=pltpu.CompilerParams(use_tc_tiling_on_sc=False),
)
def _deg_kernel(dst_hbm, out_hbm, idx_d, ones_v, buf_v, acc_sh):
    c = lax.axis_index("c")
    s = lax.axis_index("s")
    w = c * NS + s
    zeros16 = jnp.zeros((16,), jnp.float32)
    ones16 = jnp.ones((16,), jnp.float32)

    def fill(j, _):
        ones_v[j, pl.ds(0, DW)] = ones16
        return 0

    lax.fori_loop(0, CH, fill, 0)

    def zero(j, _):
        buf_v[j, pl.ds(0, DW)] = zeros16
        return 0

    lax.fori_loop(0, STRIPE, zero, 0)
    pltpu.sync_copy(buf_v, acc_sh.at[pl.ds(s * STRIPE, STRIPE)])
    plsc.subcore_barrier()

    def edge_chunk(j, _):
        pltpu.sync_copy(dst_hbm.at[w, j], idx_d)
        pltpu.sync_copy(ones_v, acc_sh.at[idx_d], add=True)
        return 0

    lax.fori_loop(0, NCH, edge_chunk, 0)
    plsc.subcore_barrier()
    pltpu.sync_copy(acc_sh.at[pl.ds(s * STRIPE, STRIPE)], buf_v)
    pltpu.sync_copy(buf_v, out_hbm.at[c, pl.ds(s * STRIPE, STRIPE)])


@functools.partial(
    pl.kernel,
    out_type=jax.ShapeDtypeStruct((NC, N_PAD, D), jnp.float32),
    mesh=_MESH,
    scratch_types=[
        pltpu.VMEM((CH,), jnp.int32),
        pltpu.VMEM((CH,), jnp.int32),
        pltpu.VMEM((CH, D), jnp.float32),
        pltpu.VMEM((STRIPE, D), jnp.float32),
        pltpu.VMEM_SHARED((N_PAD, D), jnp.float32),
        pltpu.SemaphoreType.DMA,
    ],
    compiler_params=pltpu.CompilerParams(use_tc_tiling_on_sc=False),
)
def _scatter_kernel(g_hbm, src_hbm, dst_hbm, out_hbm,
                    idx_s, idx_d, rows_v, buf_v, acc_sh, sem):
    c = lax.axis_index("c")
    s = lax.axis_index("s")
    w = c * NS + s
    zeros16 = jnp.zeros((16,), jnp.float32)

    def zero(j, _):
        buf_v[j, pl.ds(0, 16)] = zeros16
        buf_v[j, pl.ds(16, 16)] = zeros16
        return 0

    lax.fori_loop(0, STRIPE, zero, 0)
    pltpu.sync_copy(buf_v, acc_sh.at[pl.ds(s * STRIPE, STRIPE)])
    plsc.subcore_barrier()

    def edge_chunk(j, _):
        pltpu.sync_copy(src_hbm.at[w, j], idx_s)
        pltpu.sync_copy(dst_hbm.at[w, j], idx_d)
        pltpu.async_copy(g_hbm.at[idx_s], rows_v, sem).wait()
        pltpu.sync_copy(rows_v, acc_sh.at[idx_d], add=True)
        return 0

    lax.fori_loop(0, NCH, edge_chunk, 0)
    plsc.subcore_barrier()
    pltpu.sync_copy(acc_sh.at[pl.ds(s * STRIPE, STRIPE)], buf_v)
    pltpu.sync_copy(buf_v, out_hbm.at[c, pl.ds(s * STRIPE, STRIPE)])


BLK = 400


def _g_body(x_ref, w1_ref, degp_ref, g_ref):
    deg = jnp.sum(degp_ref[...], axis=1) * (1.0 / DW) + 1.0
    dis = lax.rsqrt(deg)
    h = jnp.dot(x_ref[...], w1_ref[...], preferred_element_type=jnp.float32)
    g_ref[...] = h * dis[:, None]


def _g_table(x, w1, deg_pt):
    return pl.pallas_call(
        _g_body,
        grid=(N // BLK,),
        in_specs=[
            pl.BlockSpec((BLK, D_IN), lambda i: (i, 0)),
            pl.BlockSpec((D_IN, D), lambda i: (0, 0)),
            pl.BlockSpec((BLK, NC * DW), lambda i: (i, 0)),
        ],
        out_specs=pl.BlockSpec((BLK, D), lambda i: (i, 0)),
        out_shape=jax.ShapeDtypeStruct((N, D), jnp.float32),
    )(x, w1, deg_pt)


def _out_body(s0_ref, s1_ref, g_ref, degp_ref, b1_ref, w2_ref, b2_ref, o_ref):
    deg = jnp.sum(degp_ref[...], axis=1) * (1.0 / DW) + 1.0
    dis = lax.rsqrt(deg)
    agg = (s0_ref[...] + s1_ref[...] + g_ref[...]) * dis[:, None]
    a = jnp.maximum(agg + b1_ref[...], 0.0)
    o = jnp.dot(a, w2_ref[...], preferred_element_type=jnp.float32)
    o_ref[...] = jnp.maximum(o + b2_ref[...], 0.0)


def _final(s0, s1, g, deg_p, b1, w2, b2):
    return pl.pallas_call(
        _out_body,
        grid=(N // BLK,),
        in_specs=[
            pl.BlockSpec((BLK, D), lambda i: (i, 0)),
            pl.BlockSpec((BLK, D), lambda i: (i, 0)),
            pl.BlockSpec((BLK, D), lambda i: (i, 0)),
            pl.BlockSpec((BLK, NC * DW), lambda i: (i, 0)),
            pl.BlockSpec((1, D), lambda i: (0, 0)),
            pl.BlockSpec((D, D), lambda i: (0, 0)),
            pl.BlockSpec((1, D), lambda i: (0, 0)),
        ],
        out_specs=pl.BlockSpec((BLK, D), lambda i: (i, 0)),
        out_shape=jax.ShapeDtypeStruct((N, D), jnp.float32),
    )(s0, s1, g, deg_p, b1, w2, b2)


def kernel(x, edge_index, W_gcn, b_gcn, W_dense, b_dense):
    src = edge_index[0].astype(jnp.int32)
    dst = edge_index[1].astype(jnp.int32)

    pad = E_PAD - E
    src_p = jnp.concatenate([src, jnp.zeros((pad,), jnp.int32)]).reshape(
        NW, NCH, CH)
    dst_p = jnp.concatenate([dst, jnp.full((pad,), N, jnp.int32)]).reshape(
        NW, NCH, CH)

    d_all = _deg_kernel(dst_p)
    deg_pt = jnp.concatenate([d_all[0, :N], d_all[1, :N]], axis=1)

    g = _g_table(x, W_gcn, deg_pt)

    s_all = _scatter_kernel(g, src_p, dst_p)

    s0 = s_all[0, :N, :]
    s1 = s_all[1, :N, :]
    return _final(s0, s1, g, deg_pt,
                  b_gcn.reshape(1, D), W_dense, b_dense.reshape(1, D))


# speedup vs baseline: 28.9360x; 1.2927x over previous
import functools

import jax
import jax.numpy as jnp
from jax import lax
from jax.experimental import pallas as pl
from jax.experimental.pallas import tpu as pltpu
from jax.experimental.pallas import tpu_sc as plsc

N = 10000
E = 320000
D_IN = 128
D = 32

NC = 2
NS = 16
NW = NC * NS

CH = 128
NCH = 80
E_PAD = NW * NCH * CH
K = 8
NG = NCH // K
STRIPE = 632
N_PAD = NS * STRIPE
DW = 16

_MESH = plsc.VectorSubcoreMesh(core_axis_name="c", subcore_axis_name="s")
_SC_PARAMS = pltpu.CompilerParams(use_tc_tiling_on_sc=False)


@functools.partial(
    pl.kernel,
    out_type=jax.ShapeDtypeStruct((NC, N_PAD, DW), jnp.float32),
    mesh=_MESH,
    scratch_types=[
        pltpu.VMEM((NCH, CH), jnp.int32),
        pltpu.VMEM((CH, DW), jnp.float32),
        pltpu.VMEM((STRIPE, DW), jnp.float32),
        pltpu.VMEM_SHARED((N_PAD, DW), jnp.float32),
        pltpu.SemaphoreType.DMA,
    ],
    compiler_params=_SC_PARAMS,
)
def _deg_kernel(dst_hbm, out_hbm, idx_v, ones_v, buf_v, acc_sh, sem):
    c = lax.axis_index("c")
    s = lax.axis_index("s")
    w = c * NS + s
    zeros16 = jnp.zeros((16,), jnp.float32)
    ones16 = jnp.ones((16,), jnp.float32)

    pltpu.sync_copy(dst_hbm.at[w], idx_v)

    def fill(j, _):
        ones_v[j, pl.ds(0, DW)] = ones16
        return 0

    lax.fori_loop(0, CH, fill, 0)

    def zero(j, _):
        buf_v[j, pl.ds(0, DW)] = zeros16
        return 0

    lax.fori_loop(0, STRIPE, zero, 0)
    pltpu.sync_copy(buf_v, acc_sh.at[pl.ds(s * STRIPE, STRIPE)])
    plsc.subcore_barrier()

    def group(grp, _):
        descs = [
            pltpu.async_copy(
                ones_v, acc_sh.at[idx_v.at[grp * K + b]], sem, add=True)
            for b in range(K)
        ]
        for d_ in descs:
            d_.wait()
        return 0

    lax.fori_loop(0, NG, group, 0)
    plsc.subcore_barrier()
    pltpu.sync_copy(acc_sh.at[pl.ds(s * STRIPE, STRIPE)], buf_v)
    pltpu.sync_copy(buf_v, out_hbm.at[c, pl.ds(s * STRIPE, STRIPE)])


@functools.partial(
    pl.kernel,
    out_type=jax.ShapeDtypeStruct((NC, N_PAD, D), jnp.float32),
    mesh=_MESH,
    scratch_types=[
        pltpu.VMEM((NCH, CH), jnp.int32),
        pltpu.VMEM((NCH, CH), jnp.int32),
        pltpu.VMEM((K, CH, D), jnp.float32),
        pltpu.VMEM((STRIPE, D), jnp.float32),
        pltpu.VMEM_SHARED((N_PAD, D), jnp.float32),
        pltpu.SemaphoreType.DMA,
        pltpu.SemaphoreType.DMA,
    ],
    compiler_params=_SC_PARAMS,
)
def _scatter_kernel(g_hbm, src_hbm, dst_hbm, out_hbm,
                    idx_s, idx_d, rows_v, buf_v, acc_sh, gsem, ssem):
    c = lax.axis_index("c")
    s = lax.axis_index("s")
    w = c * NS + s
    zeros16 = jnp.zeros((16,), jnp.float32)

    pltpu.sync_copy(src_hbm.at[w], idx_s)
    pltpu.sync_copy(dst_hbm.at[w], idx_d)

    def zero(j, _):
        buf_v[j, pl.ds(0, 16)] = zeros16
        buf_v[j, pl.ds(16, 16)] = zeros16
        return 0

    lax.fori_loop(0, STRIPE, zero, 0)
    pltpu.sync_copy(buf_v, acc_sh.at[pl.ds(s * STRIPE, STRIPE)])
    plsc.subcore_barrier()

    def group(grp, _):
        gets = [
            pltpu.async_copy(
                g_hbm.at[idx_s.at[grp * K + b]], rows_v.at[b], gsem)
            for b in range(K)
        ]
        for d_ in gets:
            d_.wait()
        puts = [
            pltpu.async_copy(
                rows_v.at[b], acc_sh.at[idx_d.at[grp * K + b]], ssem,
                add=True)
            for b in range(K)
        ]
        for d_ in puts:
            d_.wait()
        return 0

    lax.fori_loop(0, NG, group, 0)
    plsc.subcore_barrier()
    pltpu.sync_copy(acc_sh.at[pl.ds(s * STRIPE, STRIPE)], buf_v)
    pltpu.sync_copy(buf_v, out_hbm.at[c, pl.ds(s * STRIPE, STRIPE)])


BLK = 400


def _g_body(x_ref, w1_ref, d0_ref, d1_ref, g_ref):
    cnt = jnp.sum(d0_ref[0], axis=1) + jnp.sum(d1_ref[0], axis=1)
    deg = cnt * (1.0 / DW) + 1.0
    dis = lax.rsqrt(deg)
    h = jnp.dot(x_ref[...], w1_ref[...], preferred_element_type=jnp.float32)
    g_ref[...] = h * dis[:, None]


def _g_table(x, w1, d_all):
    return pl.pallas_call(
        _g_body,
        grid=(N // BLK,),
        in_specs=[
            pl.BlockSpec((BLK, D_IN), lambda i: (i, 0)),
            pl.BlockSpec((D_IN, D), lambda i: (0, 0)),
            pl.BlockSpec((1, BLK, DW), lambda i: (0, i, 0)),
            pl.BlockSpec((1, BLK, DW), lambda i: (1, i, 0)),
        ],
        out_specs=pl.BlockSpec((BLK, D), lambda i: (i, 0)),
        out_shape=jax.ShapeDtypeStruct((N, D), jnp.float32),
    )(x, w1, d_all, d_all)


def _out_body(s0_ref, s1_ref, g_ref, d0_ref, d1_ref, b1_ref, w2_ref, b2_ref,
              o_ref):
    cnt = jnp.sum(d0_ref[0], axis=1) + jnp.sum(d1_ref[0], axis=1)
    deg = cnt * (1.0 / DW) + 1.0
    dis = lax.rsqrt(deg)
    agg = (s0_ref[0] + s1_ref[0] + g_ref[...]) * dis[:, None]
    a = jnp.maximum(agg + b1_ref[...], 0.0)
    o = jnp.dot(a, w2_ref[...], preferred_element_type=jnp.float32)
    o_ref[...] = jnp.maximum(o + b2_ref[...], 0.0)


def _final(s_all, g, d_all, b1, w2, b2):
    return pl.pallas_call(
        _out_body,
        grid=(N // BLK,),
        in_specs=[
            pl.BlockSpec((1, BLK, D), lambda i: (0, i, 0)),
            pl.BlockSpec((1, BLK, D), lambda i: (1, i, 0)),
            pl.BlockSpec((BLK, D), lambda i: (i, 0)),
            pl.BlockSpec((1, BLK, DW), lambda i: (0, i, 0)),
            pl.BlockSpec((1, BLK, DW), lambda i: (1, i, 0)),
            pl.BlockSpec((1, D), lambda i: (0, 0)),
            pl.BlockSpec((D, D), lambda i: (0, 0)),
            pl.BlockSpec((1, D), lambda i: (0, 0)),
        ],
        out_specs=pl.BlockSpec((BLK, D), lambda i: (i, 0)),
        out_shape=jax.ShapeDtypeStruct((N, D), jnp.float32),
    )(s_all, s_all, g, d_all, d_all, b1, w2, b2)


def kernel(x, edge_index, W_gcn, b_gcn, W_dense, b_dense):
    src = edge_index[0].astype(jnp.int32)
    dst = edge_index[1].astype(jnp.int32)

    pad = E_PAD - E
    src_p = jnp.concatenate([src, jnp.zeros((pad,), jnp.int32)]).reshape(
        NW, NCH, CH)
    dst_p = jnp.concatenate([dst, jnp.full((pad,), N, jnp.int32)]).reshape(
        NW, NCH, CH)

    d_all = _deg_kernel(dst_p)
    g = _g_table(x, W_gcn, d_all)
    s_all = _scatter_kernel(g, src_p, dst_p)
    return _final(s_all, g, d_all,
                  b_gcn.reshape(1, D), W_dense, b_dense.reshape(1, D))


# speedup vs baseline: 33.1731x; 1.1464x over previous
import functools

import jax
import jax.numpy as jnp
from jax import lax
from jax.experimental import pallas as pl
from jax.experimental.pallas import tpu as pltpu
from jax.experimental.pallas import tpu_sc as plsc

N = 10000
E = 320000
D_IN = 128
D = 32

NC = 2
NS = 16
NW = NC * NS

CH = 128
NCH = 80
E_PAD = NW * NCH * CH
K = 8
NG = NCH // K
STRIPE = 632
N_PAD = NS * STRIPE
DW = 16

_MESH = plsc.VectorSubcoreMesh(core_axis_name="c", subcore_axis_name="s")
_SC_PARAMS = pltpu.CompilerParams(use_tc_tiling_on_sc=False)


@functools.partial(
    pl.kernel,
    out_type=jax.ShapeDtypeStruct((NC, N_PAD, DW), jnp.float32),
    mesh=_MESH,
    scratch_types=[
        pltpu.VMEM((NCH, CH), jnp.int32),
        pltpu.VMEM((CH, DW), jnp.float32),
        pltpu.VMEM((STRIPE, DW), jnp.float32),
        pltpu.VMEM_SHARED((N_PAD, DW), jnp.float32),
        pltpu.SemaphoreType.DMA,
    ],
    compiler_params=_SC_PARAMS,
)
def _deg_kernel(dst_hbm, out_hbm, idx_v, ones_v, buf_v, acc_sh, sem):
    c = lax.axis_index("c")
    s = lax.axis_index("s")
    w = c * NS + s
    zeros16 = jnp.zeros((16,), jnp.float32)
    ones16 = jnp.ones((16,), jnp.float32)

    pltpu.sync_copy(dst_hbm.at[w], idx_v)

    def fill(j, _):
        ones_v[j, pl.ds(0, DW)] = ones16
        return 0

    lax.fori_loop(0, CH, fill, 0)

    def zero(j, _):
        buf_v[j, pl.ds(0, DW)] = zeros16
        return 0

    lax.fori_loop(0, STRIPE, zero, 0)
    pltpu.sync_copy(buf_v, acc_sh.at[pl.ds(s * STRIPE, STRIPE)])
    plsc.subcore_barrier()

    def group(grp, _):
        descs = [
            pltpu.async_copy(
                ones_v, acc_sh.at[idx_v.at[grp * K + b]], sem, add=True)
            for b in range(K)
        ]
        for d_ in descs:
            d_.wait()
        return 0

    lax.fori_loop(0, NG, group, 0)
    plsc.subcore_barrier()
    pltpu.sync_copy(acc_sh.at[pl.ds(s * STRIPE, STRIPE)], buf_v)
    pltpu.sync_copy(buf_v, out_hbm.at[c, pl.ds(s * STRIPE, STRIPE)])


@functools.partial(
    pl.kernel,
    out_type=jax.ShapeDtypeStruct((NC, N_PAD, D), jnp.float32),
    mesh=_MESH,
    scratch_types=[
        pltpu.VMEM((NCH, CH), jnp.int32),
        pltpu.VMEM((NCH, CH), jnp.int32),
        pltpu.VMEM((K, CH, D), jnp.float32),
        pltpu.VMEM((STRIPE, D), jnp.float32),
        pltpu.VMEM_SHARED((N_PAD, D), jnp.float32),
        pltpu.SemaphoreType.DMA,
        pltpu.SemaphoreType.DMA,
    ],
    compiler_params=_SC_PARAMS,
)
def _scatter_kernel(g_hbm, src_hbm, dst_hbm, out_hbm,
                    idx_s, idx_d, rows_v, buf_v, acc_sh, gsem, ssem):
    c = lax.axis_index("c")
    s = lax.axis_index("s")
    w = c * NS + s
    zeros16 = jnp.zeros((16,), jnp.float32)

    pltpu.sync_copy(src_hbm.at[w], idx_s)
    pltpu.sync_copy(dst_hbm.at[w], idx_d)

    def zero(j, _):
        buf_v[j, pl.ds(0, 16)] = zeros16
        buf_v[j, pl.ds(16, 16)] = zeros16
        return 0

    lax.fori_loop(0, STRIPE, zero, 0)
    pltpu.sync_copy(buf_v, acc_sh.at[pl.ds(s * STRIPE, STRIPE)])
    plsc.subcore_barrier()

    def group(grp, _):
        gets = [
            pltpu.async_copy(
                g_hbm.at[idx_s.at[grp * K + b]], rows_v.at[b], gsem)
            for b in range(K)
        ]
        for d_ in gets:
            d_.wait()
        puts = [
            pltpu.async_copy(
                rows_v.at[b], acc_sh.at[idx_d.at[grp * K + b]], ssem,
                add=True)
            for b in range(K)
        ]
        for d_ in puts:
            d_.wait()
        return 0

    lax.fori_loop(0, NG, group, 0)
    plsc.subcore_barrier()
    pltpu.sync_copy(acc_sh.at[pl.ds(s * STRIPE, STRIPE)], buf_v)
    pltpu.sync_copy(buf_v, out_hbm.at[c, pl.ds(s * STRIPE, STRIPE)])


BLK = 2000


def _g_body(x_ref, w1_ref, d0_ref, d1_ref, g_ref):
    cnt = jnp.sum(d0_ref[0], axis=1) + jnp.sum(d1_ref[0], axis=1)
    deg = cnt * (1.0 / DW) + 1.0
    dis = lax.rsqrt(deg)
    h = jnp.dot(x_ref[...], w1_ref[...], preferred_element_type=jnp.float32)
    g_ref[...] = h * dis[:, None]


def _g_table(x, w1, d_all):
    return pl.pallas_call(
        _g_body,
        grid=(N // BLK,),
        in_specs=[
            pl.BlockSpec((BLK, D_IN), lambda i: (i, 0)),
            pl.BlockSpec((D_IN, D), lambda i: (0, 0)),
            pl.BlockSpec((1, BLK, DW), lambda i: (0, i, 0)),
            pl.BlockSpec((1, BLK, DW), lambda i: (1, i, 0)),
        ],
        out_specs=pl.BlockSpec((BLK, D), lambda i: (i, 0)),
        out_shape=jax.ShapeDtypeStruct((N, D), jnp.float32),
    )(x, w1, d_all, d_all)


def _out_body(s0_ref, s1_ref, g_ref, d0_ref, d1_ref, b1_ref, w2_ref, b2_ref,
              o_ref):
    cnt = jnp.sum(d0_ref[0], axis=1) + jnp.sum(d1_ref[0], axis=1)
    deg = cnt * (1.0 / DW) + 1.0
    dis = lax.rsqrt(deg)
    agg = (s0_ref[0] + s1_ref[0] + g_ref[...]) * dis[:, None]
    a = jnp.maximum(agg + b1_ref[...], 0.0)
    o = jnp.dot(a, w2_ref[...], preferred_element_type=jnp.float32)
    o_ref[...] = jnp.maximum(o + b2_ref[...], 0.0)


def _final(s_all, g, d_all, b1, w2, b2):
    return pl.pallas_call(
        _out_body,
        grid=(N // BLK,),
        in_specs=[
            pl.BlockSpec((1, BLK, D), lambda i: (0, i, 0)),
            pl.BlockSpec((1, BLK, D), lambda i: (1, i, 0)),
            pl.BlockSpec((BLK, D), lambda i: (i, 0)),
            pl.BlockSpec((1, BLK, DW), lambda i: (0, i, 0)),
            pl.BlockSpec((1, BLK, DW), lambda i: (1, i, 0)),
            pl.BlockSpec((1, D), lambda i: (0, 0)),
            pl.BlockSpec((D, D), lambda i: (0, 0)),
            pl.BlockSpec((1, D), lambda i: (0, 0)),
        ],
        out_specs=pl.BlockSpec((BLK, D), lambda i: (i, 0)),
        out_shape=jax.ShapeDtypeStruct((N, D), jnp.float32),
    )(s_all, s_all, g, d_all, d_all, b1, w2, b2)


def kernel(x, edge_index, W_gcn, b_gcn, W_dense, b_dense):
    src = edge_index[0].astype(jnp.int32)
    dst = edge_index[1].astype(jnp.int32)

    pad = E_PAD - E
    pad_dst = N + (jnp.arange(pad, dtype=jnp.int32) % (N_PAD - N))
    src_p = jnp.concatenate([src, jnp.zeros((pad,), jnp.int32)]).reshape(
        NW, NCH, CH)
    dst_p = jnp.concatenate([dst, pad_dst]).reshape(NW, NCH, CH)

    d_all = _deg_kernel(dst_p)
    g = _g_table(x, W_gcn, d_all)
    s_all = _scatter_kernel(g, src_p, dst_p)
    return _final(s_all, g, d_all,
                  b_gcn.reshape(1, D), W_dense, b_dense.reshape(1, D))


# speedup vs baseline: 51.9015x; 1.5646x over previous
import functools

import jax
import jax.numpy as jnp
from jax import lax
from jax.experimental import pallas as pl
from jax.experimental.pallas import tpu as pltpu
from jax.experimental.pallas import tpu_sc as plsc

N = 10000
E = 320000
D_IN = 128
D = 32

NC = 2
NS = 16
NW = NC * NS

CH = 128
NCH = 80
E_PAD = NW * NCH * CH
K = 8
NG = NCH // K
STRIPE = 632
N_PAD = NS * STRIPE
DW = 16

_MESH = plsc.VectorSubcoreMesh(core_axis_name="c", subcore_axis_name="s")
_SC_PARAMS = pltpu.CompilerParams(use_tc_tiling_on_sc=False)


@functools.partial(
    pl.kernel,
    out_type=jax.ShapeDtypeStruct((NC, N_PAD, DW), jnp.float32),
    mesh=_MESH,
    scratch_types=[
        pltpu.VMEM((NCH, CH), jnp.int32),
        pltpu.VMEM((CH, DW), jnp.float32),
        pltpu.VMEM((STRIPE, DW), jnp.float32),
        pltpu.VMEM_SHARED((N_PAD, DW), jnp.float32),
        pltpu.SemaphoreType.DMA,
    ],
    compiler_params=_SC_PARAMS,
)
def _deg_kernel(dst_hbm, out_hbm, idx_v, ones_v, buf_v, acc_sh, sem):
    c = lax.axis_index("c")
    s = lax.axis_index("s")
    w = c * NS + s
    zeros16 = jnp.zeros((16,), jnp.float32)
    ones16 = jnp.ones((16,), jnp.float32)

    pltpu.sync_copy(dst_hbm.at[w], idx_v)

    def fill(j, _):
        ones_v[j, pl.ds(0, DW)] = ones16
        return 0

    lax.fori_loop(0, CH, fill, 0)

    def zero(j, _):
        buf_v[j, pl.ds(0, DW)] = zeros16
        return 0

    lax.fori_loop(0, STRIPE, zero, 0)
    pltpu.sync_copy(buf_v, acc_sh.at[pl.ds(s * STRIPE, STRIPE)])
    plsc.subcore_barrier()

    def group(grp, _):
        descs = [
            pltpu.async_copy(
                ones_v, acc_sh.at[idx_v.at[grp * K + b]], sem, add=True)
            for b in range(K)
        ]
        for d_ in descs:
            d_.wait()
        return 0

    lax.fori_loop(0, NG, group, 0)
    plsc.subcore_barrier()
    pltpu.sync_copy(acc_sh.at[pl.ds(s * STRIPE, STRIPE)], buf_v)
    pltpu.sync_copy(buf_v, out_hbm.at[c, pl.ds(s * STRIPE, STRIPE)])


@functools.partial(
    pl.kernel,
    out_type=jax.ShapeDtypeStruct((NC, N_PAD, D), jnp.float32),
    mesh=_MESH,
    scratch_types=[
        pltpu.VMEM((NCH, CH), jnp.int32),
        pltpu.VMEM((NCH, CH), jnp.int32),
        pltpu.VMEM((K, CH, D), jnp.float32),
        pltpu.VMEM((STRIPE, D), jnp.float32),
        pltpu.VMEM_SHARED((N_PAD, D), jnp.float32),
        pltpu.VMEM_SHARED((N, D), jnp.float32),
        pltpu.SemaphoreType.DMA,
        pltpu.SemaphoreType.DMA,
    ],
    compiler_params=_SC_PARAMS,
)
def _scatter_kernel(g_hbm, src_hbm, dst_hbm, out_hbm,
                    idx_s, idx_d, rows_v, buf_v, acc_sh, g_sh, gsem, ssem):
    c = lax.axis_index("c")
    s = lax.axis_index("s")
    w = c * NS + s
    zeros16 = jnp.zeros((16,), jnp.float32)

    pltpu.sync_copy(src_hbm.at[w], idx_s)
    pltpu.sync_copy(dst_hbm.at[w], idx_d)

    gs = N // NS
    pltpu.sync_copy(g_hbm.at[pl.ds(s * gs, gs)], buf_v.at[pl.ds(0, gs)])
    pltpu.sync_copy(buf_v.at[pl.ds(0, gs)], g_sh.at[pl.ds(s * gs, gs)])

    def zero(j, _):
        buf_v[j, pl.ds(0, 16)] = zeros16
        buf_v[j, pl.ds(16, 16)] = zeros16
        return 0

    lax.fori_loop(0, STRIPE, zero, 0)
    pltpu.sync_copy(buf_v, acc_sh.at[pl.ds(s * STRIPE, STRIPE)])
    plsc.subcore_barrier()

    def group(grp, _):
        gets = [
            pltpu.async_copy(
                g_sh.at[idx_s.at[grp * K + b]], rows_v.at[b], gsem)
            for b in range(K)
        ]
        for d_ in gets:
            d_.wait()
        puts = [
            pltpu.async_copy(
                rows_v.at[b], acc_sh.at[idx_d.at[grp * K + b]], ssem,
                add=True)
            for b in range(K)
        ]
        for d_ in puts:
            d_.wait()
        return 0

    lax.fori_loop(0, NG, group, 0)
    plsc.subcore_barrier()
    pltpu.sync_copy(acc_sh.at[pl.ds(s * STRIPE, STRIPE)], buf_v)
    pltpu.sync_copy(buf_v, out_hbm.at[c, pl.ds(s * STRIPE, STRIPE)])


BLK = 2000


def _g_body(x_ref, w1_ref, d0_ref, d1_ref, g_ref):
    cnt = jnp.sum(d0_ref[0], axis=1) + jnp.sum(d1_ref[0], axis=1)
    deg = cnt * (1.0 / DW) + 1.0
    dis = lax.rsqrt(deg)
    h = jnp.dot(x_ref[...], w1_ref[...], preferred_element_type=jnp.float32)
    g_ref[...] = h * dis[:, None]


def _g_table(x, w1, d_all):
    return pl.pallas_call(
        _g_body,
        grid=(N // BLK,),
        in_specs=[
            pl.BlockSpec((BLK, D_IN), lambda i: (i, 0)),
            pl.BlockSpec((D_IN, D), lambda i: (0, 0)),
            pl.BlockSpec((1, BLK, DW), lambda i: (0, i, 0)),
            pl.BlockSpec((1, BLK, DW), lambda i: (1, i, 0)),
        ],
        out_specs=pl.BlockSpec((BLK, D), lambda i: (i, 0)),
        out_shape=jax.ShapeDtypeStruct((N, D), jnp.float32),
    )(x, w1, d_all, d_all)


def _out_body(s0_ref, s1_ref, g_ref, d0_ref, d1_ref, b1_ref, w2_ref, b2_ref,
              o_ref):
    cnt = jnp.sum(d0_ref[0], axis=1) + jnp.sum(d1_ref[0], axis=1)
    deg = cnt * (1.0 / DW) + 1.0
    dis = lax.rsqrt(deg)
    agg = (s0_ref[0] + s1_ref[0] + g_ref[...]) * dis[:, None]
    a = jnp.maximum(agg + b1_ref[...], 0.0)
    o = jnp.dot(a, w2_ref[...], preferred_element_type=jnp.float32)
    o_ref[...] = jnp.maximum(o + b2_ref[...], 0.0)


def _final(s_all, g, d_all, b1, w2, b2):
    return pl.pallas_call(
        _out_body,
        grid=(N // BLK,),
        in_specs=[
            pl.BlockSpec((1, BLK, D), lambda i: (0, i, 0)),
            pl.BlockSpec((1, BLK, D), lambda i: (1, i, 0)),
            pl.BlockSpec((BLK, D), lambda i: (i, 0)),
            pl.BlockSpec((1, BLK, DW), lambda i: (0, i, 0)),
            pl.BlockSpec((1, BLK, DW), lambda i: (1, i, 0)),
            pl.BlockSpec((1, D), lambda i: (0, 0)),
            pl.BlockSpec((D, D), lambda i: (0, 0)),
            pl.BlockSpec((1, D), lambda i: (0, 0)),
        ],
        out_specs=pl.BlockSpec((BLK, D), lambda i: (i, 0)),
        out_shape=jax.ShapeDtypeStruct((N, D), jnp.float32),
    )(s_all, s_all, g, d_all, d_all, b1, w2, b2)


def kernel(x, edge_index, W_gcn, b_gcn, W_dense, b_dense):
    src = edge_index[0].astype(jnp.int32)
    dst = edge_index[1].astype(jnp.int32)

    pad = E_PAD - E
    pad_dst = N + (jnp.arange(pad, dtype=jnp.int32) % (N_PAD - N))
    src_p = jnp.concatenate([src, jnp.zeros((pad,), jnp.int32)]).reshape(
        NW, NCH, CH)
    dst_p = jnp.concatenate([dst, pad_dst]).reshape(NW, NCH, CH)

    d_all = _deg_kernel(dst_p)
    g = _g_table(x, W_gcn, d_all)
    s_all = _scatter_kernel(g, src_p, dst_p)
    return _final(s_all, g, d_all,
                  b_gcn.reshape(1, D), W_dense, b_dense.reshape(1, D))


# speedup vs baseline: 58.4421x; 1.1260x over previous
import functools

import jax
import jax.numpy as jnp
from jax import lax
from jax.experimental import pallas as pl
from jax.experimental.pallas import tpu as pltpu
from jax.experimental.pallas import tpu_sc as plsc

N = 10000
E = 320000
D_IN = 128
D = 32

NC = 2
NS = 16
NW = NC * NS
EPW = E // NW

CH = 128
NCH = 79
SLAB = NCH * CH
K = 8
NG = NCH // K
TAIL = NCH - NG * K
STRIPE = 632
N_PAD = NS * STRIPE
DW = 16

_MESH = plsc.VectorSubcoreMesh(core_axis_name="c", subcore_axis_name="s")
_SC_PARAMS = pltpu.CompilerParams(use_tc_tiling_on_sc=False)


@functools.partial(
    pl.kernel,
    out_type=[
        jax.ShapeDtypeStruct((NC, N_PAD, DW), jnp.float32),
        jax.ShapeDtypeStruct((NW, NCH, CH), jnp.int32),
        jax.ShapeDtypeStruct((NW, NCH, CH), jnp.int32),
    ],
    mesh=_MESH,
    scratch_types=[
        pltpu.VMEM((SLAB,), jnp.int32),
        pltpu.VMEM((SLAB,), jnp.int32),
        pltpu.VMEM((NCH, CH), jnp.int32),
        pltpu.VMEM((NCH, CH), jnp.int32),
        pltpu.VMEM((CH, DW), jnp.float32),
        pltpu.VMEM((STRIPE, DW), jnp.float32),
        pltpu.VMEM_SHARED((N_PAD, DW), jnp.float32),
        pltpu.SemaphoreType.DMA,
    ],
    compiler_params=_SC_PARAMS,
)
def _deg_kernel(ei_hbm, out_hbm, src_o, dst_o,
                flat_s, flat_d, id2_s, id2_d, ones_v, buf_v, acc_sh, sem):
    c = lax.axis_index("c")
    s = lax.axis_index("s")
    w = c * NS + s
    zeros16 = jnp.zeros((16,), jnp.float32)
    ones16 = jnp.ones((16,), jnp.float32)
    zeros16i = jnp.zeros((16,), jnp.int32)
    iota16 = lax.iota(jnp.int32, 16)

    pltpu.sync_copy(ei_hbm.at[pl.ds(w * EPW, EPW)], flat_s.at[pl.ds(0, EPW)])
    pltpu.sync_copy(ei_hbm.at[pl.ds(E + w * EPW, EPW)],
                    flat_d.at[pl.ds(0, EPW)])
    for t in range((SLAB - EPW) // 16):
        flat_s[pl.ds(EPW + t * 16, 16)] = zeros16i
        flat_d[pl.ds(EPW + t * 16, 16)] = iota16 + (N + t * 16)

    def repack(j, _):
        for t in range(CH // 16):
            id2_s[j, pl.ds(t * 16, 16)] = flat_s[pl.ds(j * CH + t * 16, 16)]
            id2_d[j, pl.ds(t * 16, 16)] = flat_d[pl.ds(j * CH + t * 16, 16)]
        return 0

    lax.fori_loop(0, NCH, repack, 0)
    pltpu.sync_copy(id2_s, src_o.at[w])
    pltpu.sync_copy(id2_d, dst_o.at[w])

    def fill(j, _):
        ones_v[j, pl.ds(0, DW)] = ones16
        return 0

    lax.fori_loop(0, CH, fill, 0)

    def zero(j, _):
        buf_v[j, pl.ds(0, DW)] = zeros16
        return 0

    lax.fori_loop(0, STRIPE, zero, 0)
    pltpu.sync_copy(buf_v, acc_sh.at[pl.ds(s * STRIPE, STRIPE)])
    plsc.subcore_barrier()

    def group(grp, _):
        descs = [
            pltpu.async_copy(
                ones_v, acc_sh.at[id2_d.at[grp * K + b]], sem, add=True)
            for b in range(K)
        ]
        for d_ in descs:
            d_.wait()
        return 0

    lax.fori_loop(0, NG, group, 0)
    tails = [
        pltpu.async_copy(
            ones_v, acc_sh.at[id2_d.at[NG * K + b]], sem, add=True)
        for b in range(TAIL)
    ]
    for d_ in tails:
        d_.wait()
    plsc.subcore_barrier()
    pltpu.sync_copy(acc_sh.at[pl.ds(s * STRIPE, STRIPE)], buf_v)
    pltpu.sync_copy(buf_v, out_hbm.at[c, pl.ds(s * STRIPE, STRIPE)])


@functools.partial(
    pl.kernel,
    out_type=jax.ShapeDtypeStruct((NC, N_PAD, D), jnp.float32),
    mesh=_MESH,
    scratch_types=[
        pltpu.VMEM((NCH, CH), jnp.int32),
        pltpu.VMEM((NCH, CH), jnp.int32),
        pltpu.VMEM((K, CH, D), jnp.float32),
        pltpu.VMEM((STRIPE, D), jnp.float32),
        pltpu.VMEM_SHARED((N_PAD, D), jnp.float32),
        pltpu.VMEM_SHARED((N, D), jnp.float32),
        pltpu.SemaphoreType.DMA,
        pltpu.SemaphoreType.DMA,
    ],
    compiler_params=_SC_PARAMS,
)
def _scatter_kernel(g_hbm, src_hbm, dst_hbm, out_hbm,
                    idx_s, idx_d, rows_v, buf_v, acc_sh, g_sh, gsem, ssem):
    c = lax.axis_index("c")
    s = lax.axis_index("s")
    w = c * NS + s
    zeros16 = jnp.zeros((16,), jnp.float32)

    pltpu.sync_copy(src_hbm.at[w], idx_s)
    pltpu.sync_copy(dst_hbm.at[w], idx_d)

    gs = N // NS
    pltpu.sync_copy(g_hbm.at[pl.ds(s * gs, gs)], buf_v.at[pl.ds(0, gs)])
    pltpu.sync_copy(buf_v.at[pl.ds(0, gs)], g_sh.at[pl.ds(s * gs, gs)])

    def zero(j, _):
        buf_v[j, pl.ds(0, 16)] = zeros16
        buf_v[j, pl.ds(16, 16)] = zeros16
        return 0

    lax.fori_loop(0, STRIPE, zero, 0)
    pltpu.sync_copy(buf_v, acc_sh.at[pl.ds(s * STRIPE, STRIPE)])
    plsc.subcore_barrier()

    def do_chunks(base, nb):
        gets = [
            pltpu.async_copy(
                g_sh.at[idx_s.at[base + b]], rows_v.at[b], gsem)
            for b in range(nb)
        ]
        for d_ in gets:
            d_.wait()
        puts = [
            pltpu.async_copy(
                rows_v.at[b], acc_sh.at[idx_d.at[base + b]], ssem,
                add=True)
            for b in range(nb)
        ]
        for d_ in puts:
            d_.wait()

    def group(grp, _):
        do_chunks(grp * K, K)
        return 0

    lax.fori_loop(0, NG, group, 0)
    do_chunks(NG * K, TAIL)
    plsc.subcore_barrier()
    pltpu.sync_copy(acc_sh.at[pl.ds(s * STRIPE, STRIPE)], buf_v)
    pltpu.sync_copy(buf_v, out_hbm.at[c, pl.ds(s * STRIPE, STRIPE)])


BLK = 2000


def _g_body(x_ref, w1_ref, d0_ref, d1_ref, g_ref):
    cnt = jnp.sum(d0_ref[0], axis=1) + jnp.sum(d1_ref[0], axis=1)
    deg = cnt * (1.0 / DW) + 1.0
    dis = lax.rsqrt(deg)
    h = jnp.dot(x_ref[...], w1_ref[...], preferred_element_type=jnp.float32)
    g_ref[...] = h * dis[:, None]


def _g_table(x, w1, d_all):
    return pl.pallas_call(
        _g_body,
        grid=(N // BLK,),
        in_specs=[
            pl.BlockSpec((BLK, D_IN), lambda i: (i, 0)),
            pl.BlockSpec((D_IN, D), lambda i: (0, 0)),
            pl.BlockSpec((1, BLK, DW), lambda i: (0, i, 0)),
            pl.BlockSpec((1, BLK, DW), lambda i: (1, i, 0)),
        ],
        out_specs=pl.BlockSpec((BLK, D), lambda i: (i, 0)),
        out_shape=jax.ShapeDtypeStruct((N, D), jnp.float32),
    )(x, w1, d_all, d_all)


def _out_body(s0_ref, s1_ref, g_ref, d0_ref, d1_ref, b1_ref, w2_ref, b2_ref,
              o_ref):
    cnt = jnp.sum(d0_ref[0], axis=1) + jnp.sum(d1_ref[0], axis=1)
    deg = cnt * (1.0 / DW) + 1.0
    dis = lax.rsqrt(deg)
    agg = (s0_ref[0] + s1_ref[0] + g_ref[...]) * dis[:, None]
    a = jnp.maximum(agg + b1_ref[...], 0.0)
    o = jnp.dot(a, w2_ref[...], preferred_element_type=jnp.float32)
    o_ref[...] = jnp.maximum(o + b2_ref[...], 0.0)


def _final(s_all, g, d_all, b1, w2, b2):
    return pl.pallas_call(
        _out_body,
        grid=(N // BLK,),
        in_specs=[
            pl.BlockSpec((1, BLK, D), lambda i: (0, i, 0)),
            pl.BlockSpec((1, BLK, D), lambda i: (1, i, 0)),
            pl.BlockSpec((BLK, D), lambda i: (i, 0)),
            pl.BlockSpec((1, BLK, DW), lambda i: (0, i, 0)),
            pl.BlockSpec((1, BLK, DW), lambda i: (1, i, 0)),
            pl.BlockSpec((1, D), lambda i: (0, 0)),
            pl.BlockSpec((D, D), lambda i: (0, 0)),
            pl.BlockSpec((1, D), lambda i: (0, 0)),
        ],
        out_specs=pl.BlockSpec((BLK, D), lambda i: (i, 0)),
        out_shape=jax.ShapeDtypeStruct((N, D), jnp.float32),
    )(s_all, s_all, g, d_all, d_all, b1, w2, b2)


def kernel(x, edge_index, W_gcn, b_gcn, W_dense, b_dense):
    ei = edge_index.astype(jnp.int32).reshape(2 * E)

    d_all, src2d, dst2d = _deg_kernel(ei)
    g = _g_table(x, W_gcn, d_all)
    s_all = _scatter_kernel(g, src2d, dst2d)
    return _final(s_all, g, d_all,
                  b_gcn.reshape(1, D), W_dense, b_dense.reshape(1, D))


# speedup vs baseline: 63.8661x; 1.0928x over previous
import functools

import jax
import jax.numpy as jnp
from jax import lax
from jax.experimental import pallas as pl
from jax.experimental.pallas import tpu as pltpu
from jax.experimental.pallas import tpu_sc as plsc

N = 10000
E = 320000
D_IN = 128
D = 32

NC = 2
NS = 16
NW = NC * NS
EPW = E // NW

CH = 128
NCH = 79
SLAB = NCH * CH
K = 8
NG = NCH // K
TAIL = NCH - NG * K
STRIPE = 632
N_PAD = NS * STRIPE
DW = 16

_MESH = plsc.VectorSubcoreMesh(core_axis_name="c", subcore_axis_name="s")
_SC_PARAMS = pltpu.CompilerParams(
    use_tc_tiling_on_sc=False, needs_layout_passes=False)


@functools.partial(
    pl.kernel,
    out_type=[
        jax.ShapeDtypeStruct((NC, N_PAD), jnp.float32),
        jax.ShapeDtypeStruct((NW, NCH, CH), jnp.int32),
        jax.ShapeDtypeStruct((NW, NCH, CH), jnp.int32),
    ],
    mesh=_MESH,
    scratch_types=[
        pltpu.VMEM((SLAB,), jnp.int32),
        pltpu.VMEM((SLAB,), jnp.int32),
        pltpu.VMEM((NCH, CH), jnp.int32),
        pltpu.VMEM((NCH, CH), jnp.int32),
        pltpu.VMEM((CH, DW), jnp.float32),
        pltpu.VMEM((640, DW), jnp.float32),
        pltpu.VMEM((640,), jnp.float32),
        pltpu.VMEM_SHARED((N_PAD, DW), jnp.float32),
        pltpu.SemaphoreType.DMA,
    ],
    compiler_params=_SC_PARAMS,
)
def _deg_kernel(ei_hbm, out_hbm, src_o, dst_o,
                flat_s, flat_d, id2_s, id2_d, ones_v, buf_v, red_v, acc_sh,
                sem):
    c = lax.axis_index("c")
    s = lax.axis_index("s")
    w = c * NS + s
    zeros16 = jnp.zeros((16,), jnp.float32)
    ones16 = jnp.ones((16,), jnp.float32)
    zeros16i = jnp.zeros((16,), jnp.int32)
    iota16 = lax.iota(jnp.int32, 16)

    pltpu.sync_copy(ei_hbm.at[pl.ds(w * EPW, EPW)], flat_s.at[pl.ds(0, EPW)])
    pltpu.sync_copy(ei_hbm.at[pl.ds(E + w * EPW, EPW)],
                    flat_d.at[pl.ds(0, EPW)])
    for t in range((SLAB - EPW) // 16):
        flat_s[pl.ds(EPW + t * 16, 16)] = zeros16i
        flat_d[pl.ds(EPW + t * 16, 16)] = iota16 + (N + t * 16)

    def repack(j, _):
        for t in range(CH // 16):
            id2_s[j, pl.ds(t * 16, 16)] = flat_s[pl.ds(j * CH + t * 16, 16)]
            id2_d[j, pl.ds(t * 16, 16)] = flat_d[pl.ds(j * CH + t * 16, 16)]
        return 0

    lax.fori_loop(0, NCH, repack, 0)
    pltpu.sync_copy(id2_s, src_o.at[w])
    pltpu.sync_copy(id2_d, dst_o.at[w])

    def fill(j, _):
        ones_v[j, pl.ds(0, DW)] = ones16
        return 0

    lax.fori_loop(0, CH, fill, 0)

    def zero(j, _):
        buf_v[j, pl.ds(0, DW)] = zeros16
        return 0

    lax.fori_loop(0, 640, zero, 0)
    pltpu.sync_copy(buf_v.at[pl.ds(0, STRIPE)],
                    acc_sh.at[pl.ds(s * STRIPE, STRIPE)])
    plsc.subcore_barrier()

    def group(grp, _):
        descs = [
            pltpu.async_copy(
                ones_v, acc_sh.at[id2_d.at[grp * K + b]], sem, add=True)
            for b in range(K)
        ]
        for d_ in descs:
            d_.wait()
        return 0

    lax.fori_loop(0, NG, group, 0)
    tails = [
        pltpu.async_copy(
            ones_v, acc_sh.at[id2_d.at[NG * K + b]], sem, add=True)
        for b in range(TAIL)
    ]
    for d_ in tails:
        d_.wait()
    plsc.subcore_barrier()
    pltpu.sync_copy(acc_sh.at[pl.ds(s * STRIPE, STRIPE)],
                    buf_v.at[pl.ds(0, STRIPE)])

    def rowsum(blk, _):
        rows = lax.iota(jnp.int32, 16) + blk * 16
        acc = jnp.zeros((16,), jnp.float32)
        for t in range(DW):
            cols = jnp.full((16,), t, jnp.int32)
            acc = acc + plsc.load_gather(buf_v, [rows, cols])
        red_v[pl.ds(blk * 16, 16)] = acc
        return 0

    lax.fori_loop(0, 640 // 16, rowsum, 0)
    pltpu.sync_copy(red_v.at[pl.ds(0, STRIPE)],
                    out_hbm.at[c, pl.ds(s * STRIPE, STRIPE)])


@functools.partial(
    pl.kernel,
    out_type=jax.ShapeDtypeStruct((NC, N_PAD, D), jnp.float32),
    mesh=_MESH,
    scratch_types=[
        pltpu.VMEM((NCH, CH), jnp.int32),
        pltpu.VMEM((NCH, CH), jnp.int32),
        pltpu.VMEM((K, CH, D), jnp.float32),
        pltpu.VMEM((STRIPE, D), jnp.float32),
        pltpu.VMEM_SHARED((N_PAD, D), jnp.float32),
        pltpu.VMEM_SHARED((N, D), jnp.float32),
        pltpu.SemaphoreType.DMA,
        pltpu.SemaphoreType.DMA,
    ],
    compiler_params=_SC_PARAMS,
)
def _scatter_kernel(g_hbm, src_hbm, dst_hbm, out_hbm,
                    idx_s, idx_d, rows_v, buf_v, acc_sh, g_sh, gsem, ssem):
    c = lax.axis_index("c")
    s = lax.axis_index("s")
    w = c * NS + s
    zeros16 = jnp.zeros((16,), jnp.float32)

    pltpu.sync_copy(src_hbm.at[w], idx_s)
    pltpu.sync_copy(dst_hbm.at[w], idx_d)

    gs = N // NS
    pltpu.sync_copy(g_hbm.at[pl.ds(s * gs, gs)], buf_v.at[pl.ds(0, gs)])
    pltpu.sync_copy(buf_v.at[pl.ds(0, gs)], g_sh.at[pl.ds(s * gs, gs)])

    def zero(j, _):
        buf_v[j, pl.ds(0, 16)] = zeros16
        buf_v[j, pl.ds(16, 16)] = zeros16
        return 0

    lax.fori_loop(0, STRIPE, zero, 0)
    pltpu.sync_copy(buf_v, acc_sh.at[pl.ds(s * STRIPE, STRIPE)])
    plsc.subcore_barrier()

    def do_chunks(base, nb):
        gets = [
            pltpu.async_copy(
                g_sh.at[idx_s.at[base + b]], rows_v.at[b], gsem)
            for b in range(nb)
        ]
        for d_ in gets:
            d_.wait()
        puts = [
            pltpu.async_copy(
                rows_v.at[b], acc_sh.at[idx_d.at[base + b]], ssem,
                add=True)
            for b in range(nb)
        ]
        for d_ in puts:
            d_.wait()

    def group(grp, _):
        do_chunks(grp * K, K)
        return 0

    lax.fori_loop(0, NG, group, 0)
    do_chunks(NG * K, TAIL)
    plsc.subcore_barrier()
    pltpu.sync_copy(acc_sh.at[pl.ds(s * STRIPE, STRIPE)], buf_v)
    pltpu.sync_copy(buf_v, out_hbm.at[c, pl.ds(s * STRIPE, STRIPE)])


def _dis_from(d_ref):
    cnt = d_ref[0, :N] + d_ref[1, :N]
    deg = cnt * (1.0 / DW) + 1.0
    return lax.rsqrt(deg)


def _g_body(x_ref, w1_ref, d_ref, g_ref):
    dis = _dis_from(d_ref)
    h = jnp.dot(x_ref[...], w1_ref[...], preferred_element_type=jnp.float32)
    g_ref[...] = h * dis[:, None]


def _g_table(x, w1, d_all):
    return pl.pallas_call(
        _g_body,
        out_shape=jax.ShapeDtypeStruct((N, D), jnp.float32),
    )(x, w1, d_all)


def _out_body(s_ref, g_ref, d_ref, b1_ref, w2_ref, b2_ref, o_ref):
    dis = _dis_from(d_ref)
    agg = (s_ref[0, :N] + s_ref[1, :N] + g_ref[...]) * dis[:, None]
    a = jnp.maximum(agg + b1_ref[...], 0.0)
    o = jnp.dot(a, w2_ref[...], preferred_element_type=jnp.float32)
    o_ref[...] = jnp.maximum(o + b2_ref[...], 0.0)


def _final(s_all, g, d_all, b1, w2, b2):
    return pl.pallas_call(
        _out_body,
        out_shape=jax.ShapeDtypeStruct((N, D), jnp.float32),
    )(s_all, g, d_all, b1, w2, b2)


def kernel(x, edge_index, W_gcn, b_gcn, W_dense, b_dense):
    ei = edge_index.astype(jnp.int32).reshape(2 * E)

    d_all, src2d, dst2d = _deg_kernel(ei)
    g = _g_table(x, W_gcn, d_all)
    s_all = _scatter_kernel(g, src2d, dst2d)
    return _final(s_all, g, d_all,
                  b_gcn.reshape(1, D), W_dense, b_dense.reshape(1, D))


# speedup vs baseline: 64.9891x; 1.0176x over previous
import functools

import jax
import jax.numpy as jnp
from jax import lax
from jax.experimental import pallas as pl
from jax.experimental.pallas import tpu as pltpu
from jax.experimental.pallas import tpu_sc as plsc

N = 10000
E = 320000
D_IN = 128
D = 32

NC = 2
NS = 16
NW = NC * NS
EPW = E // NW

CH = 128
NCH = 79
SLAB = NCH * CH
K = 8
NG = NCH // K
TAIL = NCH - NG * K
STRIPE = 632
N_PAD = NS * STRIPE
DW = 8

_MESH = plsc.VectorSubcoreMesh(core_axis_name="c", subcore_axis_name="s")
_SC_PARAMS = pltpu.CompilerParams(
    use_tc_tiling_on_sc=False, needs_layout_passes=False)


@functools.partial(
    pl.kernel,
    out_type=[
        jax.ShapeDtypeStruct((NC, N_PAD), jnp.float32),
        jax.ShapeDtypeStruct((NW, NCH, CH), jnp.int32),
        jax.ShapeDtypeStruct((NW, NCH, CH), jnp.int32),
    ],
    mesh=_MESH,
    scratch_types=[
        pltpu.VMEM((SLAB,), jnp.int32),
        pltpu.VMEM((SLAB,), jnp.int32),
        pltpu.VMEM((NCH, CH), jnp.int32),
        pltpu.VMEM((NCH, CH), jnp.int32),
        pltpu.VMEM((CH, DW), jnp.float32),
        pltpu.VMEM((640, DW), jnp.float32),
        pltpu.VMEM((640,), jnp.float32),
        pltpu.VMEM_SHARED((N_PAD, DW), jnp.float32),
        pltpu.SemaphoreType.DMA,
    ],
    compiler_params=_SC_PARAMS,
)
def _deg_kernel(ei_hbm, ones_hbm, zeros_hbm, out_hbm, src_o, dst_o,
                flat_s, flat_d, id2_s, id2_d, ones_v, buf_v, red_v, acc_sh,
                sem):
    c = lax.axis_index("c")
    s = lax.axis_index("s")
    w = c * NS + s
    zeros16i = jnp.zeros((16,), jnp.int32)
    iota16 = lax.iota(jnp.int32, 16)

    pltpu.sync_copy(ones_hbm, ones_v)
    pltpu.sync_copy(zeros_hbm, buf_v)
    pltpu.sync_copy(ei_hbm.at[pl.ds(w * EPW, EPW)], flat_s.at[pl.ds(0, EPW)])
    pltpu.sync_copy(ei_hbm.at[pl.ds(E + w * EPW, EPW)],
                    flat_d.at[pl.ds(0, EPW)])
    for t in range((SLAB - EPW) // 16):
        flat_s[pl.ds(EPW + t * 16, 16)] = zeros16i
        flat_d[pl.ds(EPW + t * 16, 16)] = iota16 + (N + t * 16)

    def repack(j, _):
        for t in range(CH // 16):
            id2_s[j, pl.ds(t * 16, 16)] = flat_s[pl.ds(j * CH + t * 16, 16)]
            id2_d[j, pl.ds(t * 16, 16)] = flat_d[pl.ds(j * CH + t * 16, 16)]
        return 0

    lax.fori_loop(0, NCH, repack, 0)
    pltpu.sync_copy(id2_s, src_o.at[w])
    pltpu.sync_copy(id2_d, dst_o.at[w])
    pltpu.sync_copy(buf_v.at[pl.ds(0, STRIPE)],
                    acc_sh.at[pl.ds(s * STRIPE, STRIPE)])
    plsc.subcore_barrier()

    def group(grp, _):
        descs = [
            pltpu.async_copy(
                ones_v, acc_sh.at[id2_d.at[grp * K + b]], sem, add=True)
            for b in range(K)
        ]
        for d_ in descs:
            d_.wait()
        return 0

    lax.fori_loop(0, NG, group, 0)
    tails = [
        pltpu.async_copy(
            ones_v, acc_sh.at[id2_d.at[NG * K + b]], sem, add=True)
        for b in range(TAIL)
    ]
    for d_ in tails:
        d_.wait()
    plsc.subcore_barrier()
    pltpu.sync_copy(acc_sh.at[pl.ds(s * STRIPE, STRIPE)],
                    buf_v.at[pl.ds(0, STRIPE)])

    def rowsum(blk, _):
        rows = lax.iota(jnp.int32, 16) + blk * 16
        acc = jnp.zeros((16,), jnp.float32)
        for t in range(DW):
            cols = jnp.full((16,), t, jnp.int32)
            acc = acc + plsc.load_gather(buf_v, [rows, cols])
        red_v[pl.ds(blk * 16, 16)] = acc
        return 0

    lax.fori_loop(0, 640 // 16, rowsum, 0)
    pltpu.sync_copy(red_v.at[pl.ds(0, STRIPE)],
                    out_hbm.at[c, pl.ds(s * STRIPE, STRIPE)])


@functools.partial(
    pl.kernel,
    out_type=jax.ShapeDtypeStruct((NC, N_PAD, D), jnp.float32),
    mesh=_MESH,
    scratch_types=[
        pltpu.VMEM((NCH, CH), jnp.int32),
        pltpu.VMEM((NCH, CH), jnp.int32),
        pltpu.VMEM((K, CH, D), jnp.float32),
        pltpu.VMEM((STRIPE, D), jnp.float32),
        pltpu.VMEM_SHARED((N_PAD, D), jnp.float32),
        pltpu.VMEM_SHARED((N, D), jnp.float32),
        pltpu.SemaphoreType.DMA,
        pltpu.SemaphoreType.DMA,
    ],
    compiler_params=_SC_PARAMS,
)
def _scatter_kernel(g_hbm, src_hbm, dst_hbm, out_hbm,
                    idx_s, idx_d, rows_v, buf_v, acc_sh, g_sh, gsem, ssem):
    c = lax.axis_index("c")
    s = lax.axis_index("s")
    w = c * NS + s
    zeros16 = jnp.zeros((16,), jnp.float32)

    pltpu.sync_copy(src_hbm.at[w], idx_s)
    pltpu.sync_copy(dst_hbm.at[w], idx_d)

    gs = N // NS
    pltpu.sync_copy(g_hbm.at[pl.ds(s * gs, gs)], buf_v.at[pl.ds(0, gs)])
    pltpu.sync_copy(buf_v.at[pl.ds(0, gs)], g_sh.at[pl.ds(s * gs, gs)])

    def zero(j, _):
        buf_v[j, pl.ds(0, 16)] = zeros16
        buf_v[j, pl.ds(16, 16)] = zeros16
        return 0

    lax.fori_loop(0, STRIPE, zero, 0)
    pltpu.sync_copy(buf_v, acc_sh.at[pl.ds(s * STRIPE, STRIPE)])
    plsc.subcore_barrier()

    def do_chunks(base, nb):
        gets = [
            pltpu.async_copy(
                g_sh.at[idx_s.at[base + b]], rows_v.at[b], gsem)
            for b in range(nb)
        ]
        for d_ in gets:
            d_.wait()
        puts = [
            pltpu.async_copy(
                rows_v.at[b], acc_sh.at[idx_d.at[base + b]], ssem,
                add=True)
            for b in range(nb)
        ]
        for d_ in puts:
            d_.wait()

    def group(grp, _):
        do_chunks(grp * K, K)
        return 0

    lax.fori_loop(0, NG, group, 0)
    do_chunks(NG * K, TAIL)
    plsc.subcore_barrier()
    pltpu.sync_copy(acc_sh.at[pl.ds(s * STRIPE, STRIPE)], buf_v)
    pltpu.sync_copy(buf_v, out_hbm.at[c, pl.ds(s * STRIPE, STRIPE)])


def _dis_from(d_ref):
    cnt = d_ref[0, :N] + d_ref[1, :N]
    deg = cnt * (1.0 / DW) + 1.0
    return lax.rsqrt(deg)


def _h_body(x_ref, w1_ref, h_ref):
    h_ref[...] = jnp.dot(x_ref[...], w1_ref[...],
                         preferred_element_type=jnp.float32)


def _h_matmul(x, w1):
    return pl.pallas_call(
        _h_body,
        out_shape=jax.ShapeDtypeStruct((N, D), jnp.float32),
    )(x, w1)


def _g_body(h_ref, d_ref, g_ref):
    dis = _dis_from(d_ref)
    g_ref[...] = h_ref[...] * dis[:, None]


def _g_table(h, d_all):
    return pl.pallas_call(
        _g_body,
        out_shape=jax.ShapeDtypeStruct((N, D), jnp.float32),
    )(h, d_all)


def _out_body(s_ref, g_ref, d_ref, b1_ref, w2_ref, b2_ref, o_ref):
    dis = _dis_from(d_ref)
    agg = (s_ref[0, :N] + s_ref[1, :N] + g_ref[...]) * dis[:, None]
    a = jnp.maximum(agg + b1_ref[...], 0.0)
    o = jnp.dot(a, w2_ref[...], preferred_element_type=jnp.float32)
    o_ref[...] = jnp.maximum(o + b2_ref[...], 0.0)


def _final(s_all, g, d_all, b1, w2, b2):
    return pl.pallas_call(
        _out_body,
        out_shape=jax.ShapeDtypeStruct((N, D), jnp.float32),
    )(s_all, g, d_all, b1, w2, b2)


def kernel(x, edge_index, W_gcn, b_gcn, W_dense, b_dense):
    ei = edge_index.astype(jnp.int32).reshape(2 * E)
    ones_c = jnp.ones((CH, DW), jnp.float32)
    zeros_c = jnp.zeros((640, DW), jnp.float32)

    h = _h_matmul(x, W_gcn)
    d_all, src2d, dst2d = _deg_kernel(ei, ones_c, zeros_c)
    g = _g_table(h, d_all)
    s_all = _scatter_kernel(g, src2d, dst2d)
    return _final(s_all, g, d_all,
                  b_gcn.reshape(1, D), W_dense, b_dense.reshape(1, D))
